# baseline (speedup 1.0000x reference)
import functools

import jax
import jax.numpy as jnp
from jax import lax
from jax.experimental import pallas as pl
from jax.experimental.pallas import tpu as pltpu
from jax.experimental.pallas import tpu_sc as plsc

_B, _N = 64, 8192
_B_SC = 64
_B_TC = _B - _B_SC
_NC, _NS, _L = 2, 16, 16
_RG, _CG = 8, 4
_RPW = _B_SC // _RG
_CPW = _N // _CG
_CHUNKS = 2
_CC = _CPW // _CHUNKS
_VPR = _CC // _L
_TC_RB = 8


def _sc_body(x_hbm, out_hbm, xin_v, xout_v, insem0, insem1, outsem0, outsem1):
    insems = (insem0, insem1)
    outsems = (outsem0, outsem1)
    wid = lax.axis_index("s") * _NC + lax.axis_index("c")
    rg = wid // _CG
    cg = wid - rg * _CG
    r0 = rg * _RPW
    c0 = cg * _CPW

    iot = lax.iota(jnp.int32, _L)
    off = iot & 3
    blk = iot - off
    dnums = lax.GatherDimensionNumbers(
        offset_dims=(), collapsed_slice_dims=(0,), start_index_map=(0,))

    def shuf(v, r):
        noff = (off + r) & 3
        return lax.gather(v, (blk | noff)[:, None], dnums, (1,),
                          mode=lax.GatherScatterMode.PROMISE_IN_BOUNDS)

    zero = jnp.float32(0)

    def in_copy(k, buf):
        return pltpu.async_copy(
            x_hbm.at[pl.ds(r0, _RPW), pl.ds(c0 + k * _CC, _CC)],
            xin_v.at[buf], insems[buf])

    def out_copy(k, buf):
        return pltpu.async_copy(
            xout_v.at[buf],
            out_hbm.at[pl.ds(r0, _RPW), pl.ds(c0 + k * _CC, _CC)],
            outsems[buf])

    h_in = [None, None]
    h_out = [None, None]
    h_in[0] = in_copy(0, 0)
    h_in[1] = in_copy(1, 1)
    for k in range(_CHUNKS):
        buf = k & 1
        h_in[buf].wait()
        if k >= 2:
            h_out[buf].wait()

        def step(i, _):
            for row in range(_RPW):
                for g in range(2):
                    c = i * (2 * _L) + g * _L
                    v = xin_v[buf, row, pl.ds(c, _L)]
                    r2 = shuf(v, 2)
                    p = jnp.maximum(v, r2)
                    m = jnp.minimum(v, r2)
                    q = shuf(p, 1)
                    n = shuf(m, 1)
                    t2 = jnp.maximum(jnp.minimum(p, q), jnp.maximum(m, n))
                    xout_v[buf, row, pl.ds(c, _L)] = jnp.where(v >= t2, v, zero)
            return 0

        lax.fori_loop(0, _VPR // 2, step, 0)
        h_out[buf] = out_copy(k, buf)
        if k + 2 < _CHUNKS:
            h_in[buf] = in_copy(k + 2, buf)
    h_out[0].wait()
    h_out[1].wait()


def _sc_half(x):
    mesh = plsc.VectorSubcoreMesh(core_axis_name="c", subcore_axis_name="s")
    fn = functools.partial(
        pl.kernel,
        mesh=mesh,
        out_type=jax.ShapeDtypeStruct((_B_SC, _N), jnp.float32),
        scratch_types=[
            pltpu.VMEM((2, _RPW, _CC), jnp.float32),
            pltpu.VMEM((2, _RPW, _CC), jnp.float32),
            pltpu.SemaphoreType.DMA,
            pltpu.SemaphoreType.DMA,
            pltpu.SemaphoreType.DMA,
            pltpu.SemaphoreType.DMA,
        ],
        compiler_params=pltpu.CompilerParams(use_tc_tiling_on_sc=True),
    )(_sc_body)
    return fn(x)


def _tc_body(x_ref, o_ref):
    v = x_ref[...]
    off = lax.broadcasted_iota(jnp.int32, v.shape, 1) & 3

    def rot(a, r):
        left = pltpu.roll(a, _N - r, axis=1)
        right = pltpu.roll(a, 4 - r, axis=1)
        return jnp.where(off < 4 - r, left, right)

    r2 = rot(v, 2)
    p = jnp.maximum(v, r2)
    m = jnp.minimum(v, r2)
    q = rot(p, 1)
    n = rot(m, 1)
    t2 = jnp.maximum(jnp.minimum(p, q), jnp.maximum(m, n))
    o_ref[...] = jnp.where(v >= t2, v, jnp.float32(0))


def _tc_half(x):
    return pl.pallas_call(
        _tc_body,
        grid=(_B_TC // _TC_RB,),
        in_specs=[pl.BlockSpec((_TC_RB, _N), lambda i: (i + _B_SC // _TC_RB, 0))],
        out_specs=pl.BlockSpec((_TC_RB, _N), lambda i: (i, 0)),
        out_shape=jax.ShapeDtypeStruct((_B_TC, _N), jnp.float32),
    )(x)


@jax.jit
def kernel(x):
    return _sc_half(x)


# speedup vs baseline: 1.0400x; 1.0400x over previous
import functools

import jax
import jax.numpy as jnp
from jax import lax
from jax.experimental import pallas as pl
from jax.experimental.pallas import tpu as pltpu
from jax.experimental.pallas import tpu_sc as plsc

_B, _N = 64, 8192
_NC, _NS, _L = 2, 16, 16
_RG, _CG = 8, 4
_RPW = _B // _RG
_CPW = _N // _CG
_CHUNKS = 2
_CC = _CPW // _CHUNKS
_VPR = _CC // _L


def _sc_body(x_hbm, out_hbm, xin_v, xout_v, insem0, insem1, outsem0, outsem1):
    insems = (insem0, insem1)
    outsems = (outsem0, outsem1)
    wid = lax.axis_index("s") * _NC + lax.axis_index("c")
    rg = wid // _CG
    cg = wid - rg * _CG
    r0 = rg * _RPW
    c0 = cg * _CPW

    iot = lax.iota(jnp.int32, _L)
    off = iot & 3
    blk = iot - off
    dnums = lax.GatherDimensionNumbers(
        offset_dims=(), collapsed_slice_dims=(0,), start_index_map=(0,))

    def shuf(v, r):
        noff = (off + r) & 3
        return lax.gather(v, (blk | noff)[:, None], dnums, (1,),
                          mode=lax.GatherScatterMode.PROMISE_IN_BOUNDS)

    zero = jnp.float32(0)

    def in_copy(k, buf):
        return pltpu.async_copy(
            x_hbm.at[pl.ds(r0, _RPW), pl.ds(c0 + k * _CC, _CC)],
            xin_v.at[buf], insems[buf])

    def out_copy(k, buf):
        return pltpu.async_copy(
            xout_v.at[buf],
            out_hbm.at[pl.ds(r0, _RPW), pl.ds(c0 + k * _CC, _CC)],
            outsems[buf])

    h_in = [None, None]
    h_out = [None, None]
    h_in[0] = in_copy(0, 0)
    h_in[1] = in_copy(1, 1)
    for k in range(_CHUNKS):
        buf = k & 1
        h_in[buf].wait()
        if k >= 2:
            h_out[buf].wait()

        def step(i, _):
            c = i * _L
            for row in range(_RPW):
                v = xin_v[buf, row, pl.ds(c, _L)]
                r2 = shuf(v, 2)
                p = jnp.maximum(v, r2)
                m = jnp.minimum(v, r2)
                q = shuf(p, 1)
                n = shuf(m, 1)
                t2 = jnp.maximum(jnp.minimum(p, q), jnp.maximum(m, n))
                xout_v[buf, row, pl.ds(c, _L)] = jnp.where(v >= t2, v, zero)
            return 0

        lax.fori_loop(0, _VPR, step, 0)
        h_out[buf] = out_copy(k, buf)
        if k + 2 < _CHUNKS:
            h_in[buf] = in_copy(k + 2, buf)
    h_out[0].wait()
    h_out[1].wait()


def _sc_half(x):
    mesh = plsc.VectorSubcoreMesh(core_axis_name="c", subcore_axis_name="s")
    fn = functools.partial(
        pl.kernel,
        mesh=mesh,
        out_type=jax.ShapeDtypeStruct((_B, _N), jnp.float32),
        scratch_types=[
            pltpu.VMEM((2, _RPW, _CC), jnp.float32),
            pltpu.VMEM((2, _RPW, _CC), jnp.float32),
            pltpu.SemaphoreType.DMA,
            pltpu.SemaphoreType.DMA,
            pltpu.SemaphoreType.DMA,
            pltpu.SemaphoreType.DMA,
        ],
        compiler_params=pltpu.CompilerParams(use_tc_tiling_on_sc=True),
    )(_sc_body)
    return fn(x)


@jax.jit
def kernel(x):
    return _sc_half(x)


# speedup vs baseline: 1.0627x; 1.0218x over previous
import functools

import jax
import jax.numpy as jnp
from jax import lax
from jax.experimental import pallas as pl
from jax.experimental.pallas import tpu as pltpu
from jax.experimental.pallas import tpu_sc as plsc

_B, _N = 64, 8192
_NC, _NS, _L = 2, 16, 16
_RG, _CG = 8, 4
_RPW = _B // _RG
_CPW = _N // _CG
_CHUNKS = 2
_CC = _CPW // _CHUNKS
_VPR = _CC // _L


def _sc_body(x_hbm, out_hbm, xin_v, xout_v, insem0, insem1, outsem0, outsem1):
    insems = (insem0, insem1)
    outsems = (outsem0, outsem1)
    wid = lax.axis_index("s") * _NC + lax.axis_index("c")
    rg = wid // _CG
    cg = wid - rg * _CG
    r0 = rg * _RPW
    c0 = cg * _CPW

    iot = lax.iota(jnp.int32, _L)
    off = iot & 3
    blk = iot - off
    dnums = lax.GatherDimensionNumbers(
        offset_dims=(), collapsed_slice_dims=(0,), start_index_map=(0,))

    def shuf(v, r):
        noff = (off + r) & 3
        return lax.gather(v, (blk | noff)[:, None], dnums, (1,),
                          mode=lax.GatherScatterMode.PROMISE_IN_BOUNDS)

    zero = jnp.float32(0)

    def in_copy(k, buf):
        return pltpu.async_copy(
            x_hbm.at[pl.ds(r0, _RPW), pl.ds(c0 + k * _CC, _CC)],
            xin_v.at[buf], insems[buf])

    def out_copy(k, buf):
        return pltpu.async_copy(
            xout_v.at[buf],
            out_hbm.at[pl.ds(r0, _RPW), pl.ds(c0 + k * _CC, _CC)],
            outsems[buf])

    h_in = [None, None]
    h_out = [None, None]
    h_in[0] = in_copy(0, 0)
    h_in[1] = in_copy(1, 1)
    for k in range(_CHUNKS):
        buf = k & 1
        h_in[buf].wait()
        if k >= 2:
            h_out[buf].wait()

        @plsc.parallel_loop(0, _VPR, 1, unroll=2)
        def step(i):
            c = i * _L
            for row in range(_RPW):
                v = xin_v[buf, row, pl.ds(c, _L)]
                r2 = shuf(v, 2)
                p = jnp.maximum(v, r2)
                m = jnp.minimum(v, r2)
                q = shuf(p, 1)
                n = shuf(m, 1)
                t2 = jnp.maximum(jnp.minimum(p, q), jnp.maximum(m, n))
                xout_v[buf, row, pl.ds(c, _L)] = jnp.where(v >= t2, v, zero)
        h_out[buf] = out_copy(k, buf)
        if k + 2 < _CHUNKS:
            h_in[buf] = in_copy(k + 2, buf)
    h_out[0].wait()
    h_out[1].wait()


def _sc_half(x):
    mesh = plsc.VectorSubcoreMesh(core_axis_name="c", subcore_axis_name="s")
    fn = functools.partial(
        pl.kernel,
        mesh=mesh,
        out_type=jax.ShapeDtypeStruct((_B, _N), jnp.float32),
        scratch_types=[
            pltpu.VMEM((2, _RPW, _CC), jnp.float32),
            pltpu.VMEM((2, _RPW, _CC), jnp.float32),
            pltpu.SemaphoreType.DMA,
            pltpu.SemaphoreType.DMA,
            pltpu.SemaphoreType.DMA,
            pltpu.SemaphoreType.DMA,
        ],
        compiler_params=pltpu.CompilerParams(use_tc_tiling_on_sc=True),
    )(_sc_body)
    return fn(x)


@jax.jit
def kernel(x):
    return _sc_half(x)
